# baseline (speedup 1.0000x reference)
import functools

import jax
import jax.numpy as jnp
from jax import lax
from jax.experimental import pallas as pl
from jax.experimental.pallas import tpu as pltpu
from jax.experimental.pallas import tpu_sc as plsc

B = 1
S = 2048
DIM = 1024
DEPTH = 2
HEADS = 16
DH = 64
BUCKET = 64
NHASH = 4
MULT = 4
NB = S // BUCKET
NCHUNK = NHASH * NB
CS = S // NB
NPAIR = HEADS * NHASH
PW = 2 * DH
QR = 256
NQ = S // QR
F32 = jnp.float32
I32 = jnp.int32
BF16 = jnp.bfloat16


def _ln(x, g, b, eps=1e-5):
    mu = jnp.mean(x, axis=-1, keepdims=True)
    var = jnp.mean((x - mu) ** 2, axis=-1, keepdims=True)
    return (x - mu) / jnp.sqrt(var + eps) * g + b


def _proj_body(x_ref, g_ref, b_ref, wqk_ref, wv_ref, qk_ref, v_ref):
    h = _ln(x_ref[...], g_ref[...], b_ref[...])
    qk_ref[...] = jnp.dot(h, wqk_ref[...], preferred_element_type=F32)
    v_ref[...] = jnp.dot(h, wv_ref[...], preferred_element_type=F32)


def _proj(x, g, b, wqk, wv):
    MT = 512
    return pl.pallas_call(
        _proj_body,
        grid=(S // MT,),
        in_specs=[
            pl.BlockSpec((MT, DIM), lambda i: (i, 0)),
            pl.BlockSpec((1, DIM), lambda i: (0, 0)),
            pl.BlockSpec((1, DIM), lambda i: (0, 0)),
            pl.BlockSpec((DIM, DIM), lambda i: (0, 0)),
            pl.BlockSpec((DIM, DIM), lambda i: (0, 0)),
        ],
        out_specs=[pl.BlockSpec((MT, DIM), lambda i: (i, 0)),
                   pl.BlockSpec((MT, DIM), lambda i: (i, 0))],
        out_shape=[jax.ShapeDtypeStruct((S, DIM), F32),
                   jax.ShapeDtypeStruct((S, DIM), F32)],
    )(x, g, b, wqk, wv)


def _buckets_body(qv_ref, rot_ref, pos_ref):
    qk = qv_ref[:, :DH]
    rot = rot_ref[...]
    R = jnp.dot(qk, rot, preferred_element_type=F32)
    lane = lax.broadcasted_iota(I32, (S, NB), 1)
    SEG = 256
    NSEG = S // SEG
    tr = lax.broadcasted_iota(I32, (SEG, SEG), 0)
    tc = lax.broadcasted_iota(I32, (SEG, SEG), 1)
    T = (tr > tc).astype(F32)
    ur = lax.broadcasted_iota(I32, (NB, NB), 0)
    uc = lax.broadcasted_iota(I32, (NB, NB), 1)
    U = (ur < uc).astype(F32)
    for n in range(NHASH):
        r = R[:, n * (NB // 2):(n + 1) * (NB // 2)]
        full = jnp.concatenate([r, -r], axis=1)
        m = jnp.max(full, axis=1, keepdims=True)
        bkt = jnp.min(jnp.where(full >= m, lane, NB), axis=1)
        O = (bkt[:, None] == lane).astype(F32)
        hists = [jnp.sum(O[s * SEG:(s + 1) * SEG], axis=0, keepdims=True)
                 for s in range(NSEG)]
        seghist = jnp.concatenate(hists, axis=0)
        segpref = jnp.concatenate(
            [jnp.zeros((1, NB), F32)] +
            [jnp.sum(seghist[:s], axis=0, keepdims=True) for s in range(1, NSEG)],
            axis=0)
        tot = jnp.sum(seghist, axis=0, keepdims=True)
        boff = jnp.dot(tot, U, preferred_element_type=F32)
        pieces = []
        for s in range(NSEG):
            Os = O[s * SEG:(s + 1) * SEG]
            rank = jnp.dot(T, Os, preferred_element_type=F32)
            base = boff + segpref[s:s + 1]
            pieces.append(jnp.sum(Os * (rank + base), axis=1))
        pos = jnp.concatenate(pieces, axis=0)
        pos_ref[0, n] = pos.astype(I32)


def _buckets_pos(qv_tab, rotf):
    return pl.pallas_call(
        _buckets_body,
        grid=(HEADS,),
        in_specs=[
            pl.BlockSpec((S, PW), lambda h: (h, 0)),
            pl.BlockSpec((DH, NHASH * NB // 2), lambda h: (0, 0)),
        ],
        out_specs=pl.BlockSpec((1, NHASH, S), lambda h: (h, 0, 0)),
        out_shape=jax.ShapeDtypeStruct((HEADS, NHASH, S), I32),
    )(qv_tab, rotf)


def _sc_sort_apply(qv_tab, pos_flat):
    mesh = plsc.VectorSubcoreMesh(core_axis_name="c", subcore_axis_name="s")
    info = plsc.get_sparse_core_info()
    NW = info.num_cores * info.num_subcores
    pairs_per_w = NPAIR // NW

    @functools.partial(
        pl.kernel, mesh=mesh,
        compiler_params=pltpu.CompilerParams(needs_layout_passes=False),
        out_type=[
            jax.ShapeDtypeStruct((NPAIR, S), I32),
            jax.ShapeDtypeStruct((NPAIR * S, PW), F32),
        ],
        scratch_types=(
            [pltpu.VMEM((S,), I32),
             pltpu.VMEM((S,), I32)]
            + [pltpu.VMEM((QR,), I32) for _ in range(NQ)]
            + [pltpu.VMEM((QR, PW), F32) for _ in range(2)]
            + [pltpu.SemaphoreType.DMA, pltpu.SemaphoreType.DMA]
        ),
    )
    def k(qv_hbm, pos_hbm, st_hbm, sqv_hbm, pos_v, st_v, *rest):
        gidx = rest[:NQ]
        rows = rest[NQ:NQ + 2]
        semg, semw = rest[NQ + 2:]
        wid = lax.axis_index("s") * info.num_cores + lax.axis_index("c")
        for kk in range(pairs_per_w):
            p = wid * pairs_per_w + kk
            h = p // NHASH
            pltpu.sync_copy(pos_hbm.at[p], pos_v)

            def scat(j, carry):
                pv = pos_v[pl.ds(j * 16, 16)]
                iv = lax.iota(I32, 16) + j * 16
                plsc.store_scatter(st_v, [pv], iv)
                return carry

            lax.fori_loop(0, S // 16, scat, 0)
            st_w = pltpu.async_copy(st_v, st_hbm.at[p], semw)
            for q in range(NQ):
                def mkidx(j, carry, q=q):
                    gidx[q][pl.ds(j * 16, 16)] = (
                        st_v[pl.ds(q * QR + j * 16, 16)] + h * S)
                    return carry

                lax.fori_loop(0, QR // 16, mkidx, 0)
            gets = [None] * NQ
            puts = [None] * NQ
            gets[0] = pltpu.async_copy(qv_hbm.at[gidx[0]], rows[0], semg)
            for q in range(NQ):
                gets[q].wait()
                puts[q] = pltpu.async_copy(
                    rows[q % 2], sqv_hbm.at[pl.ds(p * S + q * QR, QR)], semw)
                if q + 1 < NQ:
                    if q >= 1:
                        puts[q - 1].wait()
                    gets[q + 1] = pltpu.async_copy(
                        qv_hbm.at[gidx[q + 1]], rows[(q + 1) % 2], semg)
            puts[NQ - 2].wait()
            puts[NQ - 1].wait()
            st_w.wait()

    return k(qv_tab, pos_flat)


def _attn_body(sqv_ref, st_ref, so_ref):
    sqk = sqv_ref[0, :, :, :DH]
    sv = sqv_ref[0, :, :, DH:]
    st = st_ref[0]
    nrm = jnp.sqrt(jnp.sum(sqk * sqk, axis=-1, keepdims=True)) + 1e-9
    nk = sqk / nrm
    kprev = jnp.concatenate([nk[-1:], nk[:-1]], axis=0)
    vprev = jnp.concatenate([sv[-1:], sv[:-1]], axis=0)
    tprev = jnp.concatenate([st[-1:], st[:-1]], axis=0)
    bk = jnp.concatenate([nk, kprev], axis=1)
    bv = jnp.concatenate([sv, vprev], axis=1)
    bkt = jnp.concatenate([st, tprev], axis=1)
    dots = lax.dot_general(sqk.astype(BF16), bk.astype(BF16),
                           (((2,), (2,)), ((0,), (0,))),
                           preferred_element_type=F32) / (DH ** 0.5)
    mask = st[:, :, None] == bkt[:, None, :]
    dots = jnp.where(mask, -1e5, dots)
    m = jnp.max(dots, axis=-1, keepdims=True)
    e = jnp.exp(dots - m)
    ssum = jnp.sum(e, axis=-1, keepdims=True)
    so = lax.dot_general((e / ssum).astype(BF16), bv.astype(BF16),
                         (((2,), (1,)), ((0,), (0,))),
                         preferred_element_type=F32)
    lse = jnp.log(ssum) + m
    pad = jnp.zeros((NCHUNK, CS, DH - 1), F32)
    so_ref[0] = jnp.concatenate([so, lse, pad], axis=-1)


def _attn(sqv, st):
    return pl.pallas_call(
        _attn_body,
        grid=(HEADS,),
        in_specs=[
            pl.BlockSpec((1, NCHUNK, CS, PW), lambda h: (h, 0, 0, 0)),
            pl.BlockSpec((1, NCHUNK, CS), lambda h: (h, 0, 0)),
        ],
        out_specs=pl.BlockSpec((1, NCHUNK, CS, PW), lambda h: (h, 0, 0, 0)),
        out_shape=jax.ShapeDtypeStruct((HEADS, NCHUNK, CS, PW), F32),
    )(sqv, st)


def _sc_gather_back(so_tab, pos_flat):
    mesh = plsc.VectorSubcoreMesh(core_axis_name="c", subcore_axis_name="s")
    info = plsc.get_sparse_core_info()
    NW = info.num_cores * info.num_subcores
    pairs_per_w = NPAIR // NW

    @functools.partial(
        pl.kernel, mesh=mesh,
        compiler_params=pltpu.CompilerParams(needs_layout_passes=False),
        out_type=jax.ShapeDtypeStruct((NPAIR * S, PW), F32),
        scratch_types=(
            [pltpu.VMEM((S,), I32)]
            + [pltpu.VMEM((QR,), I32) for _ in range(NQ)]
            + [pltpu.VMEM((QR, PW), F32) for _ in range(2)]
            + [pltpu.SemaphoreType.DMA, pltpu.SemaphoreType.DMA]
        ),
    )
    def k(so_hbm, pos_hbm, sou_hbm, pos_v, *rest):
        gidx = rest[:NQ]
        rows = rest[NQ:NQ + 2]
        semg, semw = rest[NQ + 2:]
        wid = lax.axis_index("s") * info.num_cores + lax.axis_index("c")
        for kk in range(pairs_per_w):
            p = wid * pairs_per_w + kk
            pltpu.sync_copy(pos_hbm.at[p], pos_v)
            for q in range(NQ):
                def mkidx(j, carry, q=q):
                    gidx[q][pl.ds(j * 16, 16)] = (
                        pos_v[pl.ds(q * QR + j * 16, 16)] + p * S)
                    return carry

                lax.fori_loop(0, QR // 16, mkidx, 0)
            gets = [None] * NQ
            puts = [None] * NQ
            gets[0] = pltpu.async_copy(so_hbm.at[gidx[0]], rows[0], semg)
            for q in range(NQ):
                gets[q].wait()
                puts[q] = pltpu.async_copy(
                    rows[q % 2], sou_hbm.at[pl.ds(p * S + q * QR, QR)], semw)
                if q + 1 < NQ:
                    if q >= 1:
                        puts[q - 1].wait()
                    gets[q + 1] = pltpu.async_copy(
                        so_hbm.at[gidx[q + 1]], rows[(q + 1) % 2], semg)
            puts[NQ - 2].wait()
            puts[NQ - 1].wait()

    return k(so_tab, pos_flat)


def _combine_body(sou_ref, wo_ref, bo_ref, x1_ref, y1_ref):
    sou = sou_ref[:, :, :, :DH]
    sl = sou_ref[:, :, :, DH]
    m = jnp.max(sl, axis=1, keepdims=True)
    lse = jnp.log(jnp.sum(jnp.exp(sl - m), axis=1, keepdims=True)) + m
    probs = jnp.exp(sl - lse)
    o = jnp.sum(sou * probs[:, :, :, None], axis=1)
    wo = wo_ref[...]
    acc = x1_ref[...] + bo_ref[...]
    for h in range(HEADS):
        acc = acc + jnp.dot(o[h].astype(BF16), wo[h * DH:(h + 1) * DH],
                            preferred_element_type=F32)
    y1_ref[...] = acc


def _combine(sou, wo, bo, x1):
    MT = 256
    return pl.pallas_call(
        _combine_body,
        grid=(S // MT,),
        in_specs=[
            pl.BlockSpec((HEADS, NHASH, MT, PW), lambda i: (0, 0, i, 0)),
            pl.BlockSpec((DIM, DIM), lambda i: (0, 0)),
            pl.BlockSpec((1, DIM), lambda i: (0, 0)),
            pl.BlockSpec((MT, DIM), lambda i: (i, 0)),
        ],
        out_specs=pl.BlockSpec((MT, DIM), lambda i: (i, 0)),
        out_shape=jax.ShapeDtypeStruct((S, DIM), F32),
    )(sou, wo, bo, x1)


def _ffn_body(x_ref, g_ref, b_ref, w1_ref, b1_ref, w2_ref, b2_ref, res_ref,
              y_ref):
    h = _ln(x_ref[...], g_ref[...], b_ref[...])
    a = (jnp.dot(h.astype(BF16), w1_ref[...], preferred_element_type=F32)
         + b1_ref[...])
    g = 0.5 * a * (1.0 + lax.erf(a * (2.0 ** -0.5)))
    y_ref[...] = (jnp.dot(g.astype(BF16), w2_ref[...],
                          preferred_element_type=F32)
                  + b2_ref[...] + res_ref[...])


def _ffn(x, g, b, w1, b1, w2, b2, res):
    MT = 256
    return pl.pallas_call(
        _ffn_body,
        grid=(S // MT,),
        in_specs=[
            pl.BlockSpec((MT, DIM), lambda i: (i, 0)),
            pl.BlockSpec((1, DIM), lambda i: (0, 0)),
            pl.BlockSpec((1, DIM), lambda i: (0, 0)),
            pl.BlockSpec((DIM, DIM * MULT), lambda i: (0, 0)),
            pl.BlockSpec((1, DIM * MULT), lambda i: (0, 0)),
            pl.BlockSpec((DIM * MULT, DIM), lambda i: (0, 0)),
            pl.BlockSpec((1, DIM), lambda i: (0, 0)),
            pl.BlockSpec((MT, DIM), lambda i: (i, 0)),
        ],
        out_specs=pl.BlockSpec((MT, DIM), lambda i: (i, 0)),
        out_shape=jax.ShapeDtypeStruct((S, DIM), F32),
    )(x, g, b, w1, b1, w2, b2, res)


def kernel(x, gf, bf, Wqk, Wv, Wo, bo, gg, bg, W1, b1, W2, b2, rot):
    x1 = x[0]
    x2 = x[0]
    for l in range(DEPTH):
        qk, v = _proj(x2, gf[l][None], bf[l][None], Wqk[l], Wv[l])
        rotf = rot[l].reshape(DH, NHASH * (NB // 2))
        qv_tab = jnp.concatenate(
            [qk.reshape(S, HEADS, DH), v.reshape(S, HEADS, DH)], axis=-1
        ).transpose(1, 0, 2).reshape(HEADS * S, PW)
        pos = _buckets_pos(qv_tab, rotf)
        pos_flat = pos.reshape(NPAIR, S)
        st, sqv = _sc_sort_apply(qv_tab, pos_flat)
        so = _attn(sqv.reshape(HEADS, NCHUNK, CS, PW),
                   st.reshape(HEADS, NCHUNK, CS))
        sou = _sc_gather_back(so.reshape(NPAIR * S, PW), pos_flat)
        y1 = _combine(sou.reshape(HEADS, NHASH, S, PW),
                      Wo[l].astype(BF16), bo[l][None], x1)
        y2 = _ffn(y1, gg[l][None], bg[l][None],
                  W1[l].astype(BF16), b1[l][None],
                  W2[l].astype(BF16), b2[l][None], x2)
        x1, x2 = y1, y2
    return (x1 + x2)[None]


# speedup vs baseline: 1.0703x; 1.0703x over previous
import functools

import jax
import jax.numpy as jnp
from jax import lax
from jax.experimental import pallas as pl
from jax.experimental.pallas import tpu as pltpu
from jax.experimental.pallas import tpu_sc as plsc

B = 1
S = 2048
DIM = 1024
DEPTH = 2
HEADS = 16
DH = 64
BUCKET = 64
NHASH = 4
MULT = 4
NB = S // BUCKET
NCHUNK = NHASH * NB
CS = S // NB
NPAIR = HEADS * NHASH
PW = 2 * DH
QR = 256
NQ = S // QR
F32 = jnp.float32
I32 = jnp.int32
BF16 = jnp.bfloat16


def _ln(x, g, b, eps=1e-5):
    mu = jnp.mean(x, axis=-1, keepdims=True)
    var = jnp.mean((x - mu) ** 2, axis=-1, keepdims=True)
    return (x - mu) / jnp.sqrt(var + eps) * g + b


def _proj_body(x_ref, g_ref, b_ref, wqk_ref, wv_ref, qk_ref, v_ref):
    h = _ln(x_ref[...], g_ref[...], b_ref[...])
    qk_ref[...] = jnp.dot(h, wqk_ref[...], preferred_element_type=F32)
    v_ref[...] = jnp.dot(h, wv_ref[...], preferred_element_type=F32)


def _proj(x, g, b, wqk, wv):
    MT = 512
    return pl.pallas_call(
        _proj_body,
        grid=(S // MT,),
        in_specs=[
            pl.BlockSpec((MT, DIM), lambda i: (i, 0)),
            pl.BlockSpec((1, DIM), lambda i: (0, 0)),
            pl.BlockSpec((1, DIM), lambda i: (0, 0)),
            pl.BlockSpec((DIM, DIM), lambda i: (0, 0)),
            pl.BlockSpec((DIM, DIM), lambda i: (0, 0)),
        ],
        out_specs=[pl.BlockSpec((MT, DIM), lambda i: (i, 0)),
                   pl.BlockSpec((MT, DIM), lambda i: (i, 0))],
        out_shape=[jax.ShapeDtypeStruct((S, DIM), F32),
                   jax.ShapeDtypeStruct((S, DIM), F32)],
    )(x, g, b, wqk, wv)


def _buckets_body(qv_ref, rot_ref, pos_ref):
    qk = qv_ref[:, :DH]
    rot = rot_ref[...]
    R = jnp.dot(qk, rot, preferred_element_type=F32)
    lane = lax.broadcasted_iota(I32, (S, NB), 1)
    SEG = 256
    NSEG = S // SEG
    tr = lax.broadcasted_iota(I32, (SEG, SEG), 0)
    tc = lax.broadcasted_iota(I32, (SEG, SEG), 1)
    T = (tr > tc).astype(F32)
    ur = lax.broadcasted_iota(I32, (NB, NB), 0)
    uc = lax.broadcasted_iota(I32, (NB, NB), 1)
    U = (ur < uc).astype(F32)
    for n in range(NHASH):
        r = R[:, n * (NB // 2):(n + 1) * (NB // 2)]
        full = jnp.concatenate([r, -r], axis=1)
        m = jnp.max(full, axis=1, keepdims=True)
        bkt = jnp.min(jnp.where(full >= m, lane, NB), axis=1)
        O = (bkt[:, None] == lane).astype(F32)
        hists = [jnp.sum(O[s * SEG:(s + 1) * SEG], axis=0, keepdims=True)
                 for s in range(NSEG)]
        seghist = jnp.concatenate(hists, axis=0)
        segpref = jnp.concatenate(
            [jnp.zeros((1, NB), F32)] +
            [jnp.sum(seghist[:s], axis=0, keepdims=True) for s in range(1, NSEG)],
            axis=0)
        tot = jnp.sum(seghist, axis=0, keepdims=True)
        boff = jnp.dot(tot, U, preferred_element_type=F32)
        pieces = []
        for s in range(NSEG):
            Os = O[s * SEG:(s + 1) * SEG]
            rank = jnp.dot(T, Os, preferred_element_type=F32)
            base = boff + segpref[s:s + 1]
            pieces.append(jnp.sum(Os * (rank + base), axis=1))
        pos = jnp.concatenate(pieces, axis=0)
        pos_ref[0, n] = pos.astype(I32)


def _buckets_pos(qv_tab, rotf):
    return pl.pallas_call(
        _buckets_body,
        grid=(HEADS,),
        in_specs=[
            pl.BlockSpec((S, PW), lambda h: (h, 0)),
            pl.BlockSpec((DH, NHASH * NB // 2), lambda h: (0, 0)),
        ],
        out_specs=pl.BlockSpec((1, NHASH, S), lambda h: (h, 0, 0)),
        out_shape=jax.ShapeDtypeStruct((HEADS, NHASH, S), I32),
    )(qv_tab, rotf)


def _sc_sort_apply(qv_tab, pos_flat):
    mesh = plsc.VectorSubcoreMesh(core_axis_name="c", subcore_axis_name="s")
    info = plsc.get_sparse_core_info()
    NW = info.num_cores * info.num_subcores
    pairs_per_w = NPAIR // NW

    @functools.partial(
        pl.kernel, mesh=mesh,
        compiler_params=pltpu.CompilerParams(needs_layout_passes=False),
        out_type=[
            jax.ShapeDtypeStruct((NPAIR, S), I32),
            jax.ShapeDtypeStruct((NPAIR * S, PW), F32),
        ],
        scratch_types=(
            [pltpu.VMEM((S,), I32),
             pltpu.VMEM((S,), I32)]
            + [pltpu.VMEM((QR,), I32) for _ in range(NQ)]
            + [pltpu.VMEM((QR, PW), F32) for _ in range(2)]
            + [pltpu.SemaphoreType.DMA for _ in range(5)]
        ),
    )
    def k(qv_hbm, pos_hbm, st_hbm, sqv_hbm, pos_v, st_v, *rest):
        gidx = rest[:NQ]
        rows = rest[NQ:NQ + 2]
        semg = rest[NQ + 2:NQ + 4]
        semw = rest[NQ + 4:NQ + 6]
        semst = rest[NQ + 6]
        wid = lax.axis_index("s") * info.num_cores + lax.axis_index("c")
        for kk in range(pairs_per_w):
            p = wid * pairs_per_w + kk
            h = p // NHASH
            pltpu.sync_copy(pos_hbm.at[p], pos_v)

            def scat(j, carry):
                pv = pos_v[pl.ds(j * 16, 16)]
                iv = lax.iota(I32, 16) + j * 16
                plsc.store_scatter(st_v, [pv], iv)
                return carry

            lax.fori_loop(0, S // 16, scat, 0)
            st_w = pltpu.async_copy(st_v, st_hbm.at[p], semst)
            for q in range(NQ):
                def mkidx(j, carry, q=q):
                    gidx[q][pl.ds(j * 16, 16)] = (
                        st_v[pl.ds(q * QR + j * 16, 16)] + h * S)
                    return carry

                lax.fori_loop(0, QR // 16, mkidx, 0)
            gets = [None] * NQ
            puts = [None] * NQ
            gets[0] = pltpu.async_copy(qv_hbm.at[gidx[0]], rows[0], semg[0])
            for q in range(NQ):
                gets[q].wait()
                puts[q] = pltpu.async_copy(
                    rows[q % 2], sqv_hbm.at[pl.ds(p * S + q * QR, QR)],
                    semw[q % 2])
                if q + 1 < NQ:
                    if q >= 1:
                        puts[q - 1].wait()
                    gets[q + 1] = pltpu.async_copy(
                        qv_hbm.at[gidx[q + 1]], rows[(q + 1) % 2],
                        semg[(q + 1) % 2])
            puts[NQ - 2].wait()
            puts[NQ - 1].wait()
            st_w.wait()

    return k(qv_tab, pos_flat)


def _attn_body(sqv_ref, st_ref, so_ref):
    sqk = sqv_ref[0, :, :, :DH]
    sv = sqv_ref[0, :, :, DH:]
    st = st_ref[0]
    nrm = jnp.sqrt(jnp.sum(sqk * sqk, axis=-1, keepdims=True)) + 1e-9
    nk = sqk / nrm
    kprev = jnp.concatenate([nk[-1:], nk[:-1]], axis=0)
    vprev = jnp.concatenate([sv[-1:], sv[:-1]], axis=0)
    tprev = jnp.concatenate([st[-1:], st[:-1]], axis=0)
    bk = jnp.concatenate([nk, kprev], axis=1)
    bv = jnp.concatenate([sv, vprev], axis=1)
    bkt = jnp.concatenate([st, tprev], axis=1)
    dots = lax.dot_general((sqk * (DH ** -0.5)).astype(BF16), bk.astype(BF16),
                           (((2,), (2,)), ((0,), (0,))),
                           preferred_element_type=F32)
    mask = st[:, :, None] == bkt[:, None, :]
    e = jnp.where(mask, 0.0, jnp.exp(dots))
    bv_ext = jnp.concatenate(
        [bv.astype(BF16), jnp.ones((NCHUNK, 2 * CS, 1), BF16)], axis=-1)
    so_ext = lax.dot_general(e.astype(BF16), bv_ext,
                             (((2,), (1,)), ((0,), (0,))),
                             preferred_element_type=F32)
    ssum = so_ext[:, :, DH:DH + 1]
    so_ref[0, :, :, :DH] = so_ext[:, :, :DH] / ssum
    so_ref[0, :, :, DH:DH + 1] = jnp.log(ssum)


def _attn(sqv, st):
    return pl.pallas_call(
        _attn_body,
        grid=(HEADS,),
        in_specs=[
            pl.BlockSpec((1, NCHUNK, CS, PW), lambda h: (h, 0, 0, 0)),
            pl.BlockSpec((1, NCHUNK, CS), lambda h: (h, 0, 0)),
        ],
        out_specs=pl.BlockSpec((1, NCHUNK, CS, PW), lambda h: (h, 0, 0, 0)),
        out_shape=jax.ShapeDtypeStruct((HEADS, NCHUNK, CS, PW), F32),
    )(sqv, st)


def _sc_gather_back(so_tab, pos_flat):
    mesh = plsc.VectorSubcoreMesh(core_axis_name="c", subcore_axis_name="s")
    info = plsc.get_sparse_core_info()
    NW = info.num_cores * info.num_subcores
    pairs_per_w = NPAIR // NW

    @functools.partial(
        pl.kernel, mesh=mesh,
        compiler_params=pltpu.CompilerParams(needs_layout_passes=False),
        out_type=jax.ShapeDtypeStruct((NPAIR * S, PW), F32),
        scratch_types=(
            [pltpu.VMEM((S,), I32)]
            + [pltpu.VMEM((QR,), I32) for _ in range(NQ)]
            + [pltpu.VMEM((QR, PW), F32) for _ in range(2)]
            + [pltpu.SemaphoreType.DMA for _ in range(4)]
        ),
    )
    def k(so_hbm, pos_hbm, sou_hbm, pos_v, *rest):
        gidx = rest[:NQ]
        rows = rest[NQ:NQ + 2]
        semg = rest[NQ + 2:NQ + 4]
        semw = rest[NQ + 4:NQ + 6]
        wid = lax.axis_index("s") * info.num_cores + lax.axis_index("c")
        for kk in range(pairs_per_w):
            p = wid * pairs_per_w + kk
            pltpu.sync_copy(pos_hbm.at[p], pos_v)
            for q in range(NQ):
                def mkidx(j, carry, q=q):
                    gidx[q][pl.ds(j * 16, 16)] = (
                        pos_v[pl.ds(q * QR + j * 16, 16)] + p * S)
                    return carry

                lax.fori_loop(0, QR // 16, mkidx, 0)
            gets = [None] * NQ
            puts = [None] * NQ
            gets[0] = pltpu.async_copy(so_hbm.at[gidx[0]], rows[0], semg[0])
            for q in range(NQ):
                gets[q].wait()
                puts[q] = pltpu.async_copy(
                    rows[q % 2], sou_hbm.at[pl.ds(p * S + q * QR, QR)],
                    semw[q % 2])
                if q + 1 < NQ:
                    if q >= 1:
                        puts[q - 1].wait()
                    gets[q + 1] = pltpu.async_copy(
                        so_hbm.at[gidx[q + 1]], rows[(q + 1) % 2],
                        semg[(q + 1) % 2])
            puts[NQ - 2].wait()
            puts[NQ - 1].wait()

    return k(so_tab, pos_flat)


def _combine_body(sou_ref, wo_ref, bo_ref, x1_ref, y1_ref):
    sou = sou_ref[:, :, :, :DH]
    sl = sou_ref[:, :, :, DH]
    m = jnp.max(sl, axis=1, keepdims=True)
    lse = jnp.log(jnp.sum(jnp.exp(sl - m), axis=1, keepdims=True)) + m
    probs = jnp.exp(sl - lse)
    o = jnp.sum(sou * probs[:, :, :, None], axis=1)
    oc = jnp.concatenate([o[h] for h in range(HEADS)], axis=1)
    y1_ref[...] = (jnp.dot(oc.astype(BF16), wo_ref[...],
                           preferred_element_type=F32)
                   + bo_ref[...] + x1_ref[...])


def _combine(sou, wo, bo, x1):
    MT = 256
    return pl.pallas_call(
        _combine_body,
        grid=(S // MT,),
        in_specs=[
            pl.BlockSpec((HEADS, NHASH, MT, PW), lambda i: (0, 0, i, 0)),
            pl.BlockSpec((DIM, DIM), lambda i: (0, 0)),
            pl.BlockSpec((1, DIM), lambda i: (0, 0)),
            pl.BlockSpec((MT, DIM), lambda i: (i, 0)),
        ],
        out_specs=pl.BlockSpec((MT, DIM), lambda i: (i, 0)),
        out_shape=jax.ShapeDtypeStruct((S, DIM), F32),
    )(sou, wo, bo, x1)


def _ffn_body(x_ref, g_ref, b_ref, w1_ref, b1_ref, w2_ref, b2_ref, res_ref,
              y_ref):
    h = _ln(x_ref[...], g_ref[...], b_ref[...])
    a = (jnp.dot(h.astype(BF16), w1_ref[...], preferred_element_type=F32)
         + b1_ref[...])
    g = 0.5 * a * (1.0 + lax.erf(a * (2.0 ** -0.5)))
    y_ref[...] = (jnp.dot(g.astype(BF16), w2_ref[...],
                          preferred_element_type=F32)
                  + b2_ref[...] + res_ref[...])


def _ffn(x, g, b, w1, b1, w2, b2, res):
    MT = 256
    return pl.pallas_call(
        _ffn_body,
        grid=(S // MT,),
        in_specs=[
            pl.BlockSpec((MT, DIM), lambda i: (i, 0)),
            pl.BlockSpec((1, DIM), lambda i: (0, 0)),
            pl.BlockSpec((1, DIM), lambda i: (0, 0)),
            pl.BlockSpec((DIM, DIM * MULT), lambda i: (0, 0)),
            pl.BlockSpec((1, DIM * MULT), lambda i: (0, 0)),
            pl.BlockSpec((DIM * MULT, DIM), lambda i: (0, 0)),
            pl.BlockSpec((1, DIM), lambda i: (0, 0)),
            pl.BlockSpec((MT, DIM), lambda i: (i, 0)),
        ],
        out_specs=pl.BlockSpec((MT, DIM), lambda i: (i, 0)),
        out_shape=jax.ShapeDtypeStruct((S, DIM), F32),
    )(x, g, b, w1, b1, w2, b2, res)


def kernel(x, gf, bf, Wqk, Wv, Wo, bo, gg, bg, W1, b1, W2, b2, rot):
    x1 = x[0]
    x2 = x[0]
    for l in range(DEPTH):
        qk, v = _proj(x2, gf[l][None], bf[l][None], Wqk[l], Wv[l])
        rotf = rot[l].reshape(DH, NHASH * (NB // 2))
        qv_tab = jnp.concatenate(
            [qk.reshape(S, HEADS, DH), v.reshape(S, HEADS, DH)], axis=-1
        ).transpose(1, 0, 2).reshape(HEADS * S, PW)
        pos = _buckets_pos(qv_tab, rotf)
        pos_flat = pos.reshape(NPAIR, S)
        st, sqv = _sc_sort_apply(qv_tab, pos_flat)
        so = _attn(sqv.reshape(HEADS, NCHUNK, CS, PW),
                   st.reshape(HEADS, NCHUNK, CS))
        sou = _sc_gather_back(so.reshape(NPAIR * S, PW), pos_flat)
        y1 = _combine(sou.reshape(HEADS, NHASH, S, PW),
                      Wo[l].astype(BF16), bo[l][None], x1)
        y2 = _ffn(y1, gg[l][None], bg[l][None],
                  W1[l].astype(BF16), b1[l][None],
                  W2[l].astype(BF16), b2[l][None], x2)
        x1, x2 = y1, y2
    return (x1 + x2)[None]


# speedup vs baseline: 1.3510x; 1.2622x over previous
import functools

import jax
import jax.numpy as jnp
from jax import lax
from jax.experimental import pallas as pl
from jax.experimental.pallas import tpu as pltpu
from jax.experimental.pallas import tpu_sc as plsc

B = 1
S = 2048
DIM = 1024
DEPTH = 2
HEADS = 16
DH = 64
BUCKET = 64
NHASH = 4
MULT = 4
NB = S // BUCKET
NCHUNK = NHASH * NB
CS = S // NB
NPAIR = HEADS * NHASH
PW = 2 * DH
QR = 256
NQ = S // QR
F32 = jnp.float32
I32 = jnp.int32
BF16 = jnp.bfloat16


def _ln(x, g, b, eps=1e-5):
    mu = jnp.mean(x, axis=-1, keepdims=True)
    var = jnp.mean((x - mu) ** 2, axis=-1, keepdims=True)
    return (x - mu) / jnp.sqrt(var + eps) * g + b


def _proj_body(x_ref, g_ref, b_ref, wqk_ref, wv_ref, qk_ref, v_ref):
    h = _ln(x_ref[...], g_ref[...], b_ref[...])
    qk_ref[...] = jnp.dot(h, wqk_ref[...], preferred_element_type=F32)
    v_ref[...] = jnp.dot(h, wv_ref[...], preferred_element_type=F32)


def _proj(x, g, b, wqk, wv):
    MT = 512
    return pl.pallas_call(
        _proj_body,
        grid=(S // MT,),
        in_specs=[
            pl.BlockSpec((MT, DIM), lambda i: (i, 0)),
            pl.BlockSpec((1, DIM), lambda i: (0, 0)),
            pl.BlockSpec((1, DIM), lambda i: (0, 0)),
            pl.BlockSpec((DIM, DIM), lambda i: (0, 0)),
            pl.BlockSpec((DIM, DIM), lambda i: (0, 0)),
        ],
        out_specs=[pl.BlockSpec((MT, DIM), lambda i: (i, 0)),
                   pl.BlockSpec((MT, DIM), lambda i: (i, 0))],
        out_shape=[jax.ShapeDtypeStruct((S, DIM), F32),
                   jax.ShapeDtypeStruct((S, DIM), F32)],
    )(x, g, b, wqk, wv)


def _buckets_body(qv_ref, rot_ref, pos_ref):
    qk = qv_ref[:, :DH]
    rot = rot_ref[...]
    RT = lax.dot_general(rot, qk, (((0,), (1,)), ((), ())),
                         preferred_element_type=F32)
    SEG = 256
    NSEG = S // SEG
    tr = lax.broadcasted_iota(I32, (SEG, SEG), 0)
    tc = lax.broadcasted_iota(I32, (SEG, SEG), 1)
    Tu = (tr < tc).astype(F32)
    ur = lax.broadcasted_iota(I32, (NB, NB), 0)
    uc = lax.broadcasted_iota(I32, (NB, NB), 1)
    Ul = (ur > uc).astype(F32)
    u8r = lax.broadcasted_iota(I32, (NSEG, NSEG), 0)
    u8c = lax.broadcasted_iota(I32, (NSEG, NSEG), 1)
    U8u = (u8r < u8c).astype(F32)
    ssr = lax.broadcasted_iota(I32, (S, NSEG), 0)
    ssc = lax.broadcasted_iota(I32, (S, NSEG), 1)
    SegSel = (ssr // SEG == ssc).astype(F32)
    j8r = lax.broadcasted_iota(I32, (NSEG, S), 0)
    j8c = lax.broadcasted_iota(I32, (NSEG, S), 1)
    SegSelT = (j8c // SEG == j8r).astype(F32)
    J8S = jnp.ones((NSEG, S), F32)
    for n in range(NHASH):
        rT = RT[n * (NB // 2):(n + 1) * (NB // 2)]
        fullT = jnp.concatenate([rT, -rT], axis=0)
        mT = jnp.max(fullT, axis=0, keepdims=True)
        ET = (fullT >= mT).astype(F32)
        CpreT = jnp.dot(Ul, ET, preferred_element_type=F32)
        OT = ET * (CpreT == 0.0)
        seghistT = jnp.dot(OT, SegSel, preferred_element_type=F32)
        baseT = (jnp.dot(jnp.dot(Ul, seghistT, preferred_element_type=F32),
                         J8S, preferred_element_type=F32)
                 + jnp.dot(jnp.dot(seghistT, U8u, preferred_element_type=F32),
                           SegSelT, preferred_element_type=F32))
        for s in range(NSEG):
            OTs = OT[:, s * SEG:(s + 1) * SEG]
            rankT = jnp.dot(OTs, Tu, preferred_element_type=F32)
            valT = OTs * (rankT + baseT[:, s * SEG:(s + 1) * SEG])
            pos_s = jnp.sum(valT, axis=0)
            pos_ref[0, n, pl.ds(s * SEG, SEG)] = pos_s.astype(I32)


def _buckets_pos(qv_tab, rotf):
    return pl.pallas_call(
        _buckets_body,
        grid=(HEADS,),
        in_specs=[
            pl.BlockSpec((S, PW), lambda h: (h, 0)),
            pl.BlockSpec((DH, NHASH * NB // 2), lambda h: (0, 0)),
        ],
        out_specs=pl.BlockSpec((1, NHASH, S), lambda h: (h, 0, 0)),
        out_shape=jax.ShapeDtypeStruct((HEADS, NHASH, S), I32),
    )(qv_tab, rotf)


def _sc_sort_apply(qv_tab, pos_flat):
    mesh = plsc.VectorSubcoreMesh(core_axis_name="c", subcore_axis_name="s")
    info = plsc.get_sparse_core_info()
    NW = info.num_cores * info.num_subcores
    pairs_per_w = NPAIR // NW

    @functools.partial(
        pl.kernel, mesh=mesh,
        compiler_params=pltpu.CompilerParams(needs_layout_passes=False),
        out_type=[
            jax.ShapeDtypeStruct((NPAIR, S), I32),
            jax.ShapeDtypeStruct((NPAIR * S, PW), F32),
        ],
        scratch_types=(
            [pltpu.VMEM((S,), I32),
             pltpu.VMEM((S,), I32)]
            + [pltpu.VMEM((QR,), I32) for _ in range(NQ)]
            + [pltpu.VMEM((QR, PW), F32) for _ in range(2)]
            + [pltpu.SemaphoreType.DMA for _ in range(5)]
        ),
    )
    def k(qv_hbm, pos_hbm, st_hbm, sqv_hbm, pos_v, st_v, *rest):
        gidx = rest[:NQ]
        rows = rest[NQ:NQ + 2]
        semg = rest[NQ + 2:NQ + 4]
        semw = rest[NQ + 4:NQ + 6]
        semst = rest[NQ + 6]
        wid = lax.axis_index("s") * info.num_cores + lax.axis_index("c")
        for kk in range(pairs_per_w):
            p = wid * pairs_per_w + kk
            h = p // NHASH
            pltpu.sync_copy(pos_hbm.at[p], pos_v)

            def scat(j, carry):
                pv = pos_v[pl.ds(j * 16, 16)]
                iv = lax.iota(I32, 16) + j * 16
                plsc.store_scatter(st_v, [pv], iv)
                return carry

            lax.fori_loop(0, S // 16, scat, 0)
            st_w = pltpu.async_copy(st_v, st_hbm.at[p], semst)
            for q in range(NQ):
                def mkidx(j, carry, q=q):
                    gidx[q][pl.ds(j * 16, 16)] = (
                        st_v[pl.ds(q * QR + j * 16, 16)] + h * S)
                    return carry

                lax.fori_loop(0, QR // 16, mkidx, 0)
            gets = [None] * NQ
            puts = [None] * NQ
            gets[0] = pltpu.async_copy(qv_hbm.at[gidx[0]], rows[0], semg[0])
            for q in range(NQ):
                gets[q].wait()
                puts[q] = pltpu.async_copy(
                    rows[q % 2], sqv_hbm.at[pl.ds(p * S + q * QR, QR)],
                    semw[q % 2])
                if q + 1 < NQ:
                    if q >= 1:
                        puts[q - 1].wait()
                    gets[q + 1] = pltpu.async_copy(
                        qv_hbm.at[gidx[q + 1]], rows[(q + 1) % 2],
                        semg[(q + 1) % 2])
            puts[NQ - 2].wait()
            puts[NQ - 1].wait()
            st_w.wait()

    return k(qv_tab, pos_flat)


def _attn_body(sqv_ref, st_ref, so_ref):
    sqk = sqv_ref[0, :, :, :DH]
    sv = sqv_ref[0, :, :, DH:]
    st = st_ref[0]
    nrm = jnp.sqrt(jnp.sum(sqk * sqk, axis=-1, keepdims=True)) + 1e-9
    nk = sqk / nrm
    kprev = jnp.concatenate([nk[-1:], nk[:-1]], axis=0)
    vprev = jnp.concatenate([sv[-1:], sv[:-1]], axis=0)
    tprev = jnp.concatenate([st[-1:], st[:-1]], axis=0)
    bk = jnp.concatenate([nk, kprev], axis=1)
    bv = jnp.concatenate([sv, vprev], axis=1)
    bkt = jnp.concatenate([st, tprev], axis=1)
    dots = lax.dot_general((sqk * (DH ** -0.5)).astype(BF16), bk.astype(BF16),
                           (((2,), (2,)), ((0,), (0,))),
                           preferred_element_type=F32)
    mask = st[:, :, None] == bkt[:, None, :]
    e = jnp.where(mask, 0.0, jnp.exp(dots))
    bv_ext = jnp.concatenate(
        [bv.astype(BF16), jnp.ones((NCHUNK, 2 * CS, 1), BF16)], axis=-1)
    so_ext = lax.dot_general(e.astype(BF16), bv_ext,
                             (((2,), (1,)), ((0,), (0,))),
                             preferred_element_type=F32)
    ssum = so_ext[:, :, DH:DH + 1]
    so_ref[0, :, :, :DH] = so_ext[:, :, :DH] / ssum
    so_ref[0, :, :, DH:DH + 1] = jnp.log(ssum)


def _attn(sqv, st):
    return pl.pallas_call(
        _attn_body,
        grid=(HEADS,),
        in_specs=[
            pl.BlockSpec((1, NCHUNK, CS, PW), lambda h: (h, 0, 0, 0)),
            pl.BlockSpec((1, NCHUNK, CS), lambda h: (h, 0, 0)),
        ],
        out_specs=pl.BlockSpec((1, NCHUNK, CS, PW), lambda h: (h, 0, 0, 0)),
        out_shape=jax.ShapeDtypeStruct((HEADS, NCHUNK, CS, PW), F32),
    )(sqv, st)


def _sc_gather_back(so_tab, pos_flat):
    mesh = plsc.VectorSubcoreMesh(core_axis_name="c", subcore_axis_name="s")
    info = plsc.get_sparse_core_info()
    NW = info.num_cores * info.num_subcores
    pairs_per_w = NPAIR // NW

    @functools.partial(
        pl.kernel, mesh=mesh,
        compiler_params=pltpu.CompilerParams(needs_layout_passes=False),
        out_type=jax.ShapeDtypeStruct((NPAIR * S, PW), F32),
        scratch_types=(
            [pltpu.VMEM((S,), I32)]
            + [pltpu.VMEM((QR,), I32) for _ in range(NQ)]
            + [pltpu.VMEM((QR, PW), F32) for _ in range(2)]
            + [pltpu.SemaphoreType.DMA for _ in range(4)]
        ),
    )
    def k(so_hbm, pos_hbm, sou_hbm, pos_v, *rest):
        gidx = rest[:NQ]
        rows = rest[NQ:NQ + 2]
        semg = rest[NQ + 2:NQ + 4]
        semw = rest[NQ + 4:NQ + 6]
        wid = lax.axis_index("s") * info.num_cores + lax.axis_index("c")
        for kk in range(pairs_per_w):
            p = wid * pairs_per_w + kk
            pltpu.sync_copy(pos_hbm.at[p], pos_v)
            for q in range(NQ):
                def mkidx(j, carry, q=q):
                    gidx[q][pl.ds(j * 16, 16)] = (
                        pos_v[pl.ds(q * QR + j * 16, 16)] + p * S)
                    return carry

                lax.fori_loop(0, QR // 16, mkidx, 0)
            gets = [None] * NQ
            puts = [None] * NQ
            gets[0] = pltpu.async_copy(so_hbm.at[gidx[0]], rows[0], semg[0])
            for q in range(NQ):
                gets[q].wait()
                puts[q] = pltpu.async_copy(
                    rows[q % 2], sou_hbm.at[pl.ds(p * S + q * QR, QR)],
                    semw[q % 2])
                if q + 1 < NQ:
                    if q >= 1:
                        puts[q - 1].wait()
                    gets[q + 1] = pltpu.async_copy(
                        so_hbm.at[gidx[q + 1]], rows[(q + 1) % 2],
                        semg[(q + 1) % 2])
            puts[NQ - 2].wait()
            puts[NQ - 1].wait()

    return k(so_tab, pos_flat)


def _combine_body(sou_ref, wo_ref, bo_ref, x1_ref, y1_ref):
    sou = sou_ref[:, :, :, :DH]
    sl = sou_ref[:, :, :, DH]
    m = jnp.max(sl, axis=1, keepdims=True)
    lse = jnp.log(jnp.sum(jnp.exp(sl - m), axis=1, keepdims=True)) + m
    probs = jnp.exp(sl - lse)
    o = jnp.sum(sou * probs[:, :, :, None], axis=1)
    oc = jnp.concatenate([o[h] for h in range(HEADS)], axis=1)
    y1_ref[...] = (jnp.dot(oc.astype(BF16), wo_ref[...],
                           preferred_element_type=F32)
                   + bo_ref[...] + x1_ref[...])


def _combine(sou, wo, bo, x1):
    MT = 256
    return pl.pallas_call(
        _combine_body,
        grid=(S // MT,),
        in_specs=[
            pl.BlockSpec((HEADS, NHASH, MT, PW), lambda i: (0, 0, i, 0)),
            pl.BlockSpec((DIM, DIM), lambda i: (0, 0)),
            pl.BlockSpec((1, DIM), lambda i: (0, 0)),
            pl.BlockSpec((MT, DIM), lambda i: (i, 0)),
        ],
        out_specs=pl.BlockSpec((MT, DIM), lambda i: (i, 0)),
        out_shape=jax.ShapeDtypeStruct((S, DIM), F32),
    )(sou, wo, bo, x1)


def _ffn_body(x_ref, g_ref, b_ref, w1_ref, b1_ref, w2_ref, b2_ref, res_ref,
              y_ref):
    h = _ln(x_ref[...], g_ref[...], b_ref[...])
    a = (jnp.dot(h.astype(BF16), w1_ref[...], preferred_element_type=F32)
         + b1_ref[...])
    g = 0.5 * a * (1.0 + lax.erf(a * (2.0 ** -0.5)))
    y_ref[...] = (jnp.dot(g.astype(BF16), w2_ref[...],
                          preferred_element_type=F32)
                  + b2_ref[...] + res_ref[...])


def _ffn(x, g, b, w1, b1, w2, b2, res):
    MT = 256
    return pl.pallas_call(
        _ffn_body,
        grid=(S // MT,),
        in_specs=[
            pl.BlockSpec((MT, DIM), lambda i: (i, 0)),
            pl.BlockSpec((1, DIM), lambda i: (0, 0)),
            pl.BlockSpec((1, DIM), lambda i: (0, 0)),
            pl.BlockSpec((DIM, DIM * MULT), lambda i: (0, 0)),
            pl.BlockSpec((1, DIM * MULT), lambda i: (0, 0)),
            pl.BlockSpec((DIM * MULT, DIM), lambda i: (0, 0)),
            pl.BlockSpec((1, DIM), lambda i: (0, 0)),
            pl.BlockSpec((MT, DIM), lambda i: (i, 0)),
        ],
        out_specs=pl.BlockSpec((MT, DIM), lambda i: (i, 0)),
        out_shape=jax.ShapeDtypeStruct((S, DIM), F32),
    )(x, g, b, w1, b1, w2, b2, res)


def kernel(x, gf, bf, Wqk, Wv, Wo, bo, gg, bg, W1, b1, W2, b2, rot):
    x1 = x[0]
    x2 = x[0]
    for l in range(DEPTH):
        qk, v = _proj(x2, gf[l][None], bf[l][None], Wqk[l], Wv[l])
        rotf = rot[l].reshape(DH, NHASH * (NB // 2))
        qv_tab = jnp.concatenate(
            [qk.reshape(S, HEADS, DH), v.reshape(S, HEADS, DH)], axis=-1
        ).transpose(1, 0, 2).reshape(HEADS * S, PW)
        pos = _buckets_pos(qv_tab, rotf)
        pos_flat = pos.reshape(NPAIR, S)
        st, sqv = _sc_sort_apply(qv_tab, pos_flat)
        so = _attn(sqv.reshape(HEADS, NCHUNK, CS, PW),
                   st.reshape(HEADS, NCHUNK, CS))
        sou = _sc_gather_back(so.reshape(NPAIR * S, PW), pos_flat)
        y1 = _combine(sou.reshape(HEADS, NHASH, S, PW),
                      Wo[l].astype(BF16), bo[l][None], x1)
        y2 = _ffn(y1, gg[l][None], bg[l][None],
                  W1[l].astype(BF16), b1[l][None],
                  W2[l].astype(BF16), b2[l][None], x2)
        x1, x2 = y1, y2
    return (x1 + x2)[None]


# speedup vs baseline: 1.4385x; 1.0648x over previous
import functools

import jax
import jax.numpy as jnp
from jax import lax
from jax.experimental import pallas as pl
from jax.experimental.pallas import tpu as pltpu
from jax.experimental.pallas import tpu_sc as plsc

B = 1
S = 2048
DIM = 1024
DEPTH = 2
HEADS = 16
DH = 64
BUCKET = 64
NHASH = 4
MULT = 4
NB = S // BUCKET
NCHUNK = NHASH * NB
CS = S // NB
NPAIR = HEADS * NHASH
PW = 2 * DH
QR = 256
NQ = S // QR
F32 = jnp.float32
I32 = jnp.int32
BF16 = jnp.bfloat16


def _ln(x, g, b, eps=1e-5):
    mu = jnp.mean(x, axis=-1, keepdims=True)
    var = jnp.mean((x - mu) ** 2, axis=-1, keepdims=True)
    return (x - mu) / jnp.sqrt(var + eps) * g + b


def _proj_body(x_ref, g_ref, b_ref, wqk_ref, wv_ref, qk_ref, v_ref):
    h = _ln(x_ref[...], g_ref[...], b_ref[...])
    qk_ref[...] = jnp.dot(h, wqk_ref[...], preferred_element_type=F32)
    v_ref[...] = jnp.dot(h, wv_ref[...], preferred_element_type=F32)


def _proj(x, g, b, wqk, wv):
    MT = 512
    return pl.pallas_call(
        _proj_body,
        grid=(S // MT,),
        in_specs=[
            pl.BlockSpec((MT, DIM), lambda i: (i, 0)),
            pl.BlockSpec((1, DIM), lambda i: (0, 0)),
            pl.BlockSpec((1, DIM), lambda i: (0, 0)),
            pl.BlockSpec((DIM, DIM), lambda i: (0, 0)),
            pl.BlockSpec((DIM, DIM), lambda i: (0, 0)),
        ],
        out_specs=[pl.BlockSpec((MT, DIM), lambda i: (i, 0)),
                   pl.BlockSpec((MT, DIM), lambda i: (i, 0))],
        out_shape=[jax.ShapeDtypeStruct((S, DIM), F32),
                   jax.ShapeDtypeStruct((S, DIM), F32)],
    )(x, g, b, wqk, wv)


def _buckets_body(qv_ref, rot_ref, pos_ref):
    qk = qv_ref[:, :DH]
    rot = rot_ref[...]
    RT = lax.dot_general(rot, qk, (((0,), (1,)), ((), ())),
                         preferred_element_type=F32)
    SEG = 256
    NSEG = S // SEG
    tr = lax.broadcasted_iota(I32, (SEG, SEG), 0)
    tc = lax.broadcasted_iota(I32, (SEG, SEG), 1)
    Tu = (tr < tc).astype(F32)
    ur = lax.broadcasted_iota(I32, (NB, NB), 0)
    uc = lax.broadcasted_iota(I32, (NB, NB), 1)
    Ul = (ur > uc).astype(F32)
    u8r = lax.broadcasted_iota(I32, (NSEG, NSEG), 0)
    u8c = lax.broadcasted_iota(I32, (NSEG, NSEG), 1)
    U8u = (u8r < u8c).astype(F32)
    ssr = lax.broadcasted_iota(I32, (S, NSEG), 0)
    ssc = lax.broadcasted_iota(I32, (S, NSEG), 1)
    SegSel = (ssr // SEG == ssc).astype(F32)
    j8r = lax.broadcasted_iota(I32, (NSEG, S), 0)
    j8c = lax.broadcasted_iota(I32, (NSEG, S), 1)
    SegSelT = (j8c // SEG == j8r).astype(F32)
    J8S = jnp.ones((NSEG, S), F32)
    for n in range(NHASH):
        rT = RT[n * (NB // 2):(n + 1) * (NB // 2)]
        fullT = jnp.concatenate([rT, -rT], axis=0)
        mT = jnp.max(fullT, axis=0, keepdims=True)
        ET = (fullT >= mT).astype(F32)
        CpreT = jnp.dot(Ul, ET, preferred_element_type=F32)
        OT = ET * (CpreT == 0.0)
        seghistT = jnp.dot(OT, SegSel, preferred_element_type=F32)
        baseT = (jnp.dot(jnp.dot(Ul, seghistT, preferred_element_type=F32),
                         J8S, preferred_element_type=F32)
                 + jnp.dot(jnp.dot(seghistT, U8u, preferred_element_type=F32),
                           SegSelT, preferred_element_type=F32))
        for s in range(NSEG):
            OTs = OT[:, s * SEG:(s + 1) * SEG]
            rankT = jnp.dot(OTs, Tu, preferred_element_type=F32)
            valT = OTs * (rankT + baseT[:, s * SEG:(s + 1) * SEG])
            pos_s = jnp.sum(valT, axis=0)
            pos_ref[0, n, pl.ds(s * SEG, SEG)] = pos_s.astype(I32)


def _buckets_pos(qv_tab, rotf):
    return pl.pallas_call(
        _buckets_body,
        grid=(HEADS,),
        in_specs=[
            pl.BlockSpec((S, PW), lambda h: (h, 0)),
            pl.BlockSpec((DH, NHASH * NB // 2), lambda h: (0, 0)),
        ],
        out_specs=pl.BlockSpec((1, NHASH, S), lambda h: (h, 0, 0)),
        out_shape=jax.ShapeDtypeStruct((HEADS, NHASH, S), I32),
    )(qv_tab, rotf)


def _sc_sort_apply(qv_tab, pos_flat):
    mesh = plsc.VectorSubcoreMesh(core_axis_name="c", subcore_axis_name="s")
    info = plsc.get_sparse_core_info()
    NW = info.num_cores * info.num_subcores
    pairs_per_w = NPAIR // NW

    @functools.partial(
        pl.kernel, mesh=mesh,
        compiler_params=pltpu.CompilerParams(needs_layout_passes=False),
        out_type=[
            jax.ShapeDtypeStruct((NPAIR, S), I32),
            jax.ShapeDtypeStruct((NPAIR * S, PW), F32),
        ],
        scratch_types=(
            [pltpu.VMEM((S,), I32),
             pltpu.VMEM((S,), I32)]
            + [pltpu.VMEM((QR,), I32) for _ in range(NQ)]
            + [pltpu.VMEM((QR, PW), F32) for _ in range(2)]
            + [pltpu.SemaphoreType.DMA for _ in range(5)]
        ),
    )
    def k(qv_hbm, pos_hbm, st_hbm, sqv_hbm, pos_v, st_v, *rest):
        gidx = rest[:NQ]
        rows = rest[NQ:NQ + 2]
        semg = rest[NQ + 2:NQ + 4]
        semw = rest[NQ + 4:NQ + 6]
        semst = rest[NQ + 6]
        wid = lax.axis_index("s") * info.num_cores + lax.axis_index("c")
        for kk in range(pairs_per_w):
            p = wid * pairs_per_w + kk
            h = p // NHASH
            pltpu.sync_copy(pos_hbm.at[p], pos_v)

            def scat(j, carry):
                pv = pos_v[pl.ds(j * 16, 16)]
                iv = lax.iota(I32, 16) + j * 16
                plsc.store_scatter(st_v, [pv], iv)
                return carry

            lax.fori_loop(0, S // 16, scat, 0)
            st_w = pltpu.async_copy(st_v, st_hbm.at[p], semst)
            for q in range(NQ):
                def mkidx(j, carry, q=q):
                    gidx[q][pl.ds(j * 16, 16)] = (
                        st_v[pl.ds(q * QR + j * 16, 16)] + h * S)
                    return carry

                lax.fori_loop(0, QR // 16, mkidx, 0)
            gets = [None] * NQ
            puts = [None] * NQ
            gets[0] = pltpu.async_copy(qv_hbm.at[gidx[0]], rows[0], semg[0])
            for q in range(NQ):
                gets[q].wait()
                puts[q] = pltpu.async_copy(
                    rows[q % 2], sqv_hbm.at[pl.ds(p * S + q * QR, QR)],
                    semw[q % 2])
                if q + 1 < NQ:
                    if q >= 1:
                        puts[q - 1].wait()
                    gets[q + 1] = pltpu.async_copy(
                        qv_hbm.at[gidx[q + 1]], rows[(q + 1) % 2],
                        semg[(q + 1) % 2])
            puts[NQ - 2].wait()
            puts[NQ - 1].wait()
            st_w.wait()

    return k(qv_tab, pos_flat)


def _attn_body(sqv_ref, st_ref, so_ref):
    sqk = sqv_ref[0, :, :, :DH]
    sv = sqv_ref[0, :, :, DH:]
    st = st_ref[0]
    nrm = jnp.sqrt(jnp.sum(sqk * sqk, axis=-1, keepdims=True)) + 1e-9
    nk = sqk / nrm
    kprev = jnp.concatenate([nk[-1:], nk[:-1]], axis=0)
    vprev = jnp.concatenate([sv[-1:], sv[:-1]], axis=0)
    tprev = jnp.concatenate([st[-1:], st[:-1]], axis=0)
    bk = jnp.concatenate([nk, kprev], axis=1)
    bv = jnp.concatenate([sv, vprev], axis=1)
    bkt = jnp.concatenate([st, tprev], axis=1)
    dots = lax.dot_general((sqk * (DH ** -0.5)).astype(BF16), bk.astype(BF16),
                           (((2,), (2,)), ((0,), (0,))),
                           preferred_element_type=F32)
    mask = st[:, :, None] == bkt[:, None, :]
    e = jnp.where(mask, 0.0, jnp.exp(dots))
    bv_ext = jnp.concatenate(
        [bv.astype(BF16), jnp.ones((NCHUNK, 2 * CS, 1), BF16)], axis=-1)
    so_ext = lax.dot_general(e.astype(BF16), bv_ext,
                             (((2,), (1,)), ((0,), (0,))),
                             preferred_element_type=F32)
    so_ref[0, :, :, :DH + 1] = so_ext
    so_ref[0, :, :, DH + 1:] = jnp.zeros((NCHUNK, CS, PW - DH - 1), F32)


def _attn(sqv, st):
    return pl.pallas_call(
        _attn_body,
        grid=(HEADS,),
        in_specs=[
            pl.BlockSpec((1, NCHUNK, CS, PW), lambda h: (h, 0, 0, 0)),
            pl.BlockSpec((1, NCHUNK, CS), lambda h: (h, 0, 0)),
        ],
        out_specs=pl.BlockSpec((1, NCHUNK, CS, PW), lambda h: (h, 0, 0, 0)),
        out_shape=jax.ShapeDtypeStruct((HEADS, NCHUNK, CS, PW), F32),
    )(sqv, st)


def _sc_gather_back(so_tab, pos_flat):
    mesh = plsc.VectorSubcoreMesh(core_axis_name="c", subcore_axis_name="s")
    info = plsc.get_sparse_core_info()
    NW = info.num_cores * info.num_subcores
    pairs_per_w = NPAIR // NW

    @functools.partial(
        pl.kernel, mesh=mesh,
        compiler_params=pltpu.CompilerParams(needs_layout_passes=False),
        out_type=jax.ShapeDtypeStruct((NPAIR * S, PW), F32),
        scratch_types=(
            [pltpu.VMEM((S,), I32)]
            + [pltpu.VMEM((QR,), I32) for _ in range(NQ)]
            + [pltpu.VMEM((QR, PW), F32) for _ in range(2)]
            + [pltpu.SemaphoreType.DMA for _ in range(4)]
        ),
    )
    def k(so_hbm, pos_hbm, sou_hbm, pos_v, *rest):
        gidx = rest[:NQ]
        rows = rest[NQ:NQ + 2]
        semg = rest[NQ + 2:NQ + 4]
        semw = rest[NQ + 4:NQ + 6]
        wid = lax.axis_index("s") * info.num_cores + lax.axis_index("c")
        for kk in range(pairs_per_w):
            p = wid * pairs_per_w + kk
            pltpu.sync_copy(pos_hbm.at[p], pos_v)
            for q in range(NQ):
                def mkidx(j, carry, q=q):
                    gidx[q][pl.ds(j * 16, 16)] = (
                        pos_v[pl.ds(q * QR + j * 16, 16)] + p * S)
                    return carry

                lax.fori_loop(0, QR // 16, mkidx, 0)
            gets = [None] * NQ
            puts = [None] * NQ
            gets[0] = pltpu.async_copy(so_hbm.at[gidx[0]], rows[0], semg[0])
            for q in range(NQ):
                gets[q].wait()
                puts[q] = pltpu.async_copy(
                    rows[q % 2], sou_hbm.at[pl.ds(p * S + q * QR, QR)],
                    semw[q % 2])
                if q + 1 < NQ:
                    if q >= 1:
                        puts[q - 1].wait()
                    gets[q + 1] = pltpu.async_copy(
                        so_hbm.at[gidx[q + 1]], rows[(q + 1) % 2],
                        semg[(q + 1) % 2])
            puts[NQ - 2].wait()
            puts[NQ - 1].wait()

    return k(so_tab, pos_flat)


def _combine_body(sou_ref, wo_ref, bo_ref, x1_ref, y1_ref):
    acc = jnp.sum(sou_ref[...], axis=1)
    o = acc[:, :, :DH] / acc[:, :, DH:DH + 1]
    oc = jnp.concatenate([o[h] for h in range(HEADS)], axis=1)
    y1_ref[...] = (jnp.dot(oc.astype(BF16), wo_ref[...],
                           preferred_element_type=F32)
                   + bo_ref[...] + x1_ref[...])


def _combine(sou, wo, bo, x1):
    MT = 256
    return pl.pallas_call(
        _combine_body,
        grid=(S // MT,),
        in_specs=[
            pl.BlockSpec((HEADS, NHASH, MT, PW), lambda i: (0, 0, i, 0)),
            pl.BlockSpec((DIM, DIM), lambda i: (0, 0)),
            pl.BlockSpec((1, DIM), lambda i: (0, 0)),
            pl.BlockSpec((MT, DIM), lambda i: (i, 0)),
        ],
        out_specs=pl.BlockSpec((MT, DIM), lambda i: (i, 0)),
        out_shape=jax.ShapeDtypeStruct((S, DIM), F32),
    )(sou, wo, bo, x1)


def _ffn_body(x_ref, g_ref, b_ref, w1_ref, b1_ref, w2_ref, b2_ref, res_ref,
              y_ref):
    h = _ln(x_ref[...], g_ref[...], b_ref[...])
    a = (jnp.dot(h.astype(BF16), w1_ref[...], preferred_element_type=F32)
         + b1_ref[...])
    g = 0.5 * a * (1.0 + lax.erf(a * (2.0 ** -0.5)))
    y_ref[...] = (jnp.dot(g.astype(BF16), w2_ref[...],
                          preferred_element_type=F32)
                  + b2_ref[...] + res_ref[...])


def _ffn(x, g, b, w1, b1, w2, b2, res):
    MT = 256
    return pl.pallas_call(
        _ffn_body,
        grid=(S // MT,),
        in_specs=[
            pl.BlockSpec((MT, DIM), lambda i: (i, 0)),
            pl.BlockSpec((1, DIM), lambda i: (0, 0)),
            pl.BlockSpec((1, DIM), lambda i: (0, 0)),
            pl.BlockSpec((DIM, DIM * MULT), lambda i: (0, 0)),
            pl.BlockSpec((1, DIM * MULT), lambda i: (0, 0)),
            pl.BlockSpec((DIM * MULT, DIM), lambda i: (0, 0)),
            pl.BlockSpec((1, DIM), lambda i: (0, 0)),
            pl.BlockSpec((MT, DIM), lambda i: (i, 0)),
        ],
        out_specs=pl.BlockSpec((MT, DIM), lambda i: (i, 0)),
        out_shape=jax.ShapeDtypeStruct((S, DIM), F32),
    )(x, g, b, w1, b1, w2, b2, res)


def kernel(x, gf, bf, Wqk, Wv, Wo, bo, gg, bg, W1, b1, W2, b2, rot):
    x1 = x[0]
    x2 = x[0]
    for l in range(DEPTH):
        qk, v = _proj(x2, gf[l][None], bf[l][None], Wqk[l], Wv[l])
        rotf = rot[l].reshape(DH, NHASH * (NB // 2))
        qv_tab = jnp.concatenate(
            [qk.reshape(S, HEADS, DH), v.reshape(S, HEADS, DH)], axis=-1
        ).transpose(1, 0, 2).reshape(HEADS * S, PW)
        pos = _buckets_pos(qv_tab, rotf)
        pos_flat = pos.reshape(NPAIR, S)
        st, sqv = _sc_sort_apply(qv_tab, pos_flat)
        so = _attn(sqv.reshape(HEADS, NCHUNK, CS, PW),
                   st.reshape(HEADS, NCHUNK, CS))
        sou = _sc_gather_back(so.reshape(NPAIR * S, PW), pos_flat)
        y1 = _combine(sou.reshape(HEADS, NHASH, S, PW),
                      Wo[l].astype(BF16), bo[l][None], x1)
        y2 = _ffn(y1, gg[l][None], bg[l][None],
                  W1[l].astype(BF16), b1[l][None],
                  W2[l].astype(BF16), b2[l][None], x2)
        x1, x2 = y1, y2
    return (x1 + x2)[None]
